# Initial kernel scaffold; baseline (speedup 1.0000x reference)
import jax
import jax.numpy as jnp
from jax.experimental import pallas as pl


def kernel(sparse_fea, dense_fea, sp_c1_0_W, sp_c1_0_b, sp_c1_0_g, sp_c1_0_be, sp_c1_1_W, sp_c1_1_b, sp_c1_1_g, sp_c1_1_be, sp_c2_0_W, sp_c2_0_b, sp_c2_0_g, sp_c2_0_be, sp_c2_1_W, sp_c2_1_b, sp_c2_1_g, sp_c2_1_be, sp_c3_0_W, sp_c3_0_b, sp_c3_0_g, sp_c3_0_be, sp_c3_1_W, sp_c3_1_b, sp_c3_1_g, sp_c3_1_be, dn_c1_0_W, dn_c1_0_b, dn_c1_0_g, dn_c1_0_be, dn_c1_1_W, dn_c1_1_b, dn_c1_1_g, dn_c1_1_be, dn_c2_0_W, dn_c2_0_b, dn_c2_0_g, dn_c2_0_be, dn_c2_1_W, dn_c2_1_b, dn_c2_1_g, dn_c2_1_be, dn_c3_0_W, dn_c3_0_b, dn_c3_0_g, dn_c3_0_be, dn_c3_1_W, dn_c3_1_b, dn_c3_1_g, dn_c3_1_be, d2s_W, d2s_b, d2s_g, d2s_be, ds_W, ds_b, ds_g, ds_be):
    raise NotImplementedError("write your pallas kernel here")


# speedup vs baseline: 3.4120x; 3.4120x over previous
import jax
import jax.numpy as jnp
from jax.experimental import pallas as pl

N_STROKE = 32
N_PNT = 64
N_DENSE = N_STROKE * N_PNT
KNN = 10
ROW_TILE = 256
NEG = -3.0e38
_GELU_C = 0.7978845608028654


def _gelu(x):
    cdf = 0.5 * (1.0 + jnp.tanh(_GELU_C * (x + 0.044715 * (x * x * x))))
    return x * cdf


def _bn(x, g, be):
    return x / jnp.sqrt(jnp.float32(1.0 + 1e-5)) * g + be


def _mmb(a, b):
    return jax.lax.dot_general(a.astype(jnp.bfloat16), b.astype(jnp.bfloat16),
                               (((1,), (0,)), ((), ())),
                               preferred_element_type=jnp.float32)


def _mmb_nt(a, b):
    return jax.lax.dot_general(a.astype(jnp.bfloat16), b.astype(jnp.bfloat16),
                               (((1,), (1,)), ((), ())),
                               preferred_element_type=jnp.float32)


def _mm_hi(a, b):
    return jax.lax.dot_general(a, b, (((1,), (0,)), ((), ())),
                               precision=jax.lax.Precision.HIGHEST,
                               preferred_element_type=jnp.float32)


def _shift_up(a, s):
    return jnp.concatenate([a[s:], a[:s]], axis=0)


def _shift_down(a, s):
    return jnp.concatenate([a[-s:], a[:-s]], axis=0)


def _topk_onehot_step(cur, iota_f):
    mx = jnp.max(cur, axis=1, keepdims=True)
    cand = jnp.where(cur == mx, iota_f, 3.0e38)
    amin = jnp.min(cand, axis=1, keepdims=True)
    hit = iota_f == amin
    return hit, jnp.where(hit, NEG, cur)


def _edge_mlp_step(ek, xtf, xt, w1t, b1, g1, be1, w2t, b2, g2, be2):
    feat = _mm_hi(ek, xtf)
    e = jnp.concatenate([feat - xt, xt], axis=1)
    h = _gelu(_bn(_mmb(e, w1t) + b1, g1, be1))
    return _gelu(_bn(_mmb(h, w2t) + b2, g2, be2))


def _knn_pd(xt, x):
    inner = -2.0 * _mmb(xt, x)
    xx_row = jnp.sum(x * x, axis=0, keepdims=True)
    xx_col = jnp.sum(xt * xt, axis=1, keepdims=True)
    return (-xx_col - inner) - xx_row


def _sparse_body(sft_ref, d4t_ref,
                 w0_ref, w1_ref, w2_ref, bcv_ref, gcv_ref, becv_ref,
                 w10_ref, b10_ref, g10_ref, be10_ref,
                 w11_ref, b11_ref, g11_ref, be11_ref,
                 w20_ref, b20_ref, g20_ref, be20_ref,
                 w21_ref, b21_ref, g21_ref, be21_ref,
                 w30_ref, b30_ref, g30_ref, be30_ref,
                 w31_ref, b31_ref, g31_ref, be31_ref,
                 out_ref):
    d4t = d4t_ref[0]
    a = (_mmb(d4t, w0_ref[...]) + _shift_up(_mmb(d4t, w1_ref[...]), 1)
         + _shift_up(_mmb(d4t, w2_ref[...]), 2)) + bcv_ref[...]
    z = _gelu(_bn(a, gcv_ref[...], becv_ref[...]))
    rowi = jax.lax.broadcasted_iota(jnp.int32, z.shape, 0)
    z = jnp.where((rowi & (N_PNT - 1)) < N_PNT - 2, z, NEG)
    m = z
    for s in (32, 16, 8, 4, 2, 1):
        m = jnp.maximum(m, _shift_up(m, s))
    si = jax.lax.broadcasted_iota(jnp.int32, (N_STROKE, N_DENSE), 0)
    ni = jax.lax.broadcasted_iota(jnp.int32, (N_STROKE, N_DENSE), 1)
    e1t = jnp.where(ni == si * N_PNT, 1.0, 0.0)
    ht = _mm_hi(e1t, m)
    x0 = jnp.concatenate([sft_ref[0], ht], axis=1)

    def gcn_block(xt, w1t, b1, g1, be1, w2t, b2, g2, be2):
        inner = -2.0 * _mmb_nt(xt, xt)
        sq = xt * xt
        xx_col = jnp.sum(sq, axis=1, keepdims=True)
        ones_row = jnp.ones((1, xt.shape[1]), jnp.float32)
        xx_row = jax.lax.dot_general(ones_row, sq, (((1,), (1,)), ((), ())),
                                     precision=jax.lax.Precision.HIGHEST,
                                     preferred_element_type=jnp.float32)
        pd = (-xx_col - inner) - xx_row
        iota_f = jax.lax.broadcasted_iota(jnp.int32, pd.shape, 1).astype(jnp.float32)
        cur = pd
        acc = jnp.full((xt.shape[0], w2t.shape[1]), NEG, jnp.float32)
        for _ in range(KNN):
            hit, cur = _topk_onehot_step(cur, iota_f)
            ek = jnp.where(hit, 1.0, 0.0)
            f = _edge_mlp_step(ek, xt, xt, w1t, b1, g1, be1, w2t, b2, g2, be2)
            acc = jnp.maximum(acc, f)
        return acc

    x1 = gcn_block(x0, w10_ref[...], b10_ref[...], g10_ref[...], be10_ref[...],
                   w11_ref[...], b11_ref[...], g11_ref[...], be11_ref[...])
    x2 = gcn_block(x1, w20_ref[...], b20_ref[...], g20_ref[...], be20_ref[...],
                   w21_ref[...], b21_ref[...], g21_ref[...], be21_ref[...])
    ct = jnp.concatenate([x1, x2], axis=1)
    h1 = _gelu(_bn(_mmb(ct, w30_ref[...]) + b30_ref[...],
                   g30_ref[...], be30_ref[...]))
    out_ref[0] = _gelu(_bn(_mmb(h1, w31_ref[...]) + b31_ref[...],
                           g31_ref[...], be31_ref[...]))


def _edge_body(xtf_ref, x_ref,
               w10_ref, b10_ref, g10_ref, be10_ref,
               w11_ref, b11_ref, g11_ref, be11_ref,
               out_ref):
    r = pl.program_id(1)
    xtf = xtf_ref[0]
    x = x_ref[0]
    xt = xtf_ref[0, pl.ds(r * ROW_TILE, ROW_TILE), :]
    pd = _knn_pd(xt, x)
    iota_f = jax.lax.broadcasted_iota(jnp.int32, pd.shape, 1).astype(jnp.float32)
    cur = pd
    acc = jnp.full((ROW_TILE, w11_ref.shape[1]), NEG, jnp.float32)
    for _ in range(KNN):
        hit, cur = _topk_onehot_step(cur, iota_f)
        ek = jnp.where(hit, 1.0, 0.0)
        f = _edge_mlp_step(ek, xtf, xt,
                           w10_ref[...], b10_ref[...], g10_ref[...], be10_ref[...],
                           w11_ref[...], b11_ref[...], g11_ref[...], be11_ref[...])
        acc = jnp.maximum(acc, f)
    out_ref[0] = acc


def _mlp1_body(ct_ref, w30_ref, b30_ref, g30_ref, be30_ref,
               w31_ref, b31_ref, g31_ref, be31_ref, out_ref):
    h1 = _gelu(_bn(_mmb(ct_ref[0], w30_ref[...]) + b30_ref[...],
                   g30_ref[...], be30_ref[...]))
    out_ref[0] = _gelu(_bn(_mmb(h1, w31_ref[...]) + b31_ref[...],
                           g31_ref[...], be31_ref[...]))


def _ds_body(ut_ref, w0_ref, w1_ref, w2_ref, b_ref, g_ref, be_ref, out_ref):
    ut = ut_ref[0]
    a0 = _mmb(ut, w0_ref[...])
    a1 = _mmb(ut, w1_ref[...])
    a2 = _mmb(ut, w2_ref[...])
    rowi = jax.lax.broadcasted_iota(jnp.int32, ut.shape, 0)
    pm = rowi & (N_PNT - 1)
    t0 = jnp.where(pm == 0, 0.0, _shift_down(a0, 1))
    t2 = jnp.where(pm == N_PNT - 1, 0.0, _shift_up(a2, 1))
    out_ref[0] = _gelu(_bn(t0 + a1 + t2 + b_ref[...], g_ref[...], be_ref[...]))


def _full_spec(shape):
    return pl.BlockSpec((1,) + shape[1:], lambda b, *_: (b,) + (0,) * (len(shape) - 1))


def _w_spec(shape):
    return pl.BlockSpec(shape, lambda *_: (0,) * len(shape))


def _run_batch(body, inputs, n_batched, out_shapes):
    bs = inputs[0].shape[0]
    in_specs = [_full_spec(a.shape) if i < n_batched else _w_spec(a.shape)
                for i, a in enumerate(inputs)]
    out_specs = [_full_spec((bs,) + s) for s in out_shapes]
    out_shape = [jax.ShapeDtypeStruct((bs,) + s, jnp.float32) for s in out_shapes]
    return pl.pallas_call(
        body,
        grid=(bs,),
        in_specs=in_specs,
        out_specs=out_specs if len(out_specs) > 1 else out_specs[0],
        out_shape=out_shape if len(out_shape) > 1 else out_shape[0],
    )(*inputs)


def _run_edge(xtf, x, weights):
    bs, n, c = xtf.shape
    c2 = weights[4].shape[1]
    ntile = n // ROW_TILE
    in_specs = [
        pl.BlockSpec((1, n, c), lambda b, r: (b, 0, 0)),
        pl.BlockSpec((1, c, n), lambda b, r: (b, 0, 0)),
    ] + [_w_spec(w.shape) for w in weights]
    return pl.pallas_call(
        _edge_body,
        grid=(bs, ntile),
        in_specs=in_specs,
        out_specs=pl.BlockSpec((1, ROW_TILE, c2), lambda b, r: (b, r, 0)),
        out_shape=jax.ShapeDtypeStruct((bs, n, c2), jnp.float32),
    )(xtf, x, *weights)


def _rows(v):
    return v[None, :]


def kernel(sparse_fea, dense_fea, sp_c1_0_W, sp_c1_0_b, sp_c1_0_g, sp_c1_0_be, sp_c1_1_W, sp_c1_1_b, sp_c1_1_g, sp_c1_1_be, sp_c2_0_W, sp_c2_0_b, sp_c2_0_g, sp_c2_0_be, sp_c2_1_W, sp_c2_1_b, sp_c2_1_g, sp_c2_1_be, sp_c3_0_W, sp_c3_0_b, sp_c3_0_g, sp_c3_0_be, sp_c3_1_W, sp_c3_1_b, sp_c3_1_g, sp_c3_1_be, dn_c1_0_W, dn_c1_0_b, dn_c1_0_g, dn_c1_0_be, dn_c1_1_W, dn_c1_1_b, dn_c1_1_g, dn_c1_1_be, dn_c2_0_W, dn_c2_0_b, dn_c2_0_g, dn_c2_0_be, dn_c2_1_W, dn_c2_1_b, dn_c2_1_g, dn_c2_1_be, dn_c3_0_W, dn_c3_0_b, dn_c3_0_g, dn_c3_0_be, dn_c3_1_W, dn_c3_1_b, dn_c3_1_g, dn_c3_1_be, d2s_W, d2s_b, d2s_g, d2s_be, ds_W, ds_b, ds_g, ds_be):
    bs = sparse_fea.shape[0]

    sft = jnp.transpose(sparse_fea, (0, 2, 1))
    d4t = jnp.transpose(dense_fea, (0, 2, 1))

    ust = _run_batch(
        _sparse_body,
        [sft, d4t,
         d2s_W[:, :, 0, 0].T, d2s_W[:, :, 0, 1].T, d2s_W[:, :, 0, 2].T,
         _rows(d2s_b), _rows(d2s_g), _rows(d2s_be),
         sp_c1_0_W.T, _rows(sp_c1_0_b), _rows(sp_c1_0_g), _rows(sp_c1_0_be),
         sp_c1_1_W.T, _rows(sp_c1_1_b), _rows(sp_c1_1_g), _rows(sp_c1_1_be),
         sp_c2_0_W.T, _rows(sp_c2_0_b), _rows(sp_c2_0_g), _rows(sp_c2_0_be),
         sp_c2_1_W.T, _rows(sp_c2_1_b), _rows(sp_c2_1_g), _rows(sp_c2_1_be),
         sp_c3_0_W.T, _rows(sp_c3_0_b), _rows(sp_c3_0_g), _rows(sp_c3_0_be),
         sp_c3_1_W.T, _rows(sp_c3_1_b), _rows(sp_c3_1_g), _rows(sp_c3_1_be)],
        2,
        [(N_STROKE, sp_c3_1_W.shape[0])],
    )
    us = jnp.transpose(ust, (0, 2, 1))

    sp_rep_t = jnp.repeat(sft, N_PNT, axis=1)
    xt0 = jnp.concatenate([d4t, sp_rep_t], axis=2)
    x0 = jnp.transpose(xt0, (0, 2, 1))

    w_c1 = [dn_c1_0_W.T, _rows(dn_c1_0_b), _rows(dn_c1_0_g), _rows(dn_c1_0_be),
            dn_c1_1_W.T, _rows(dn_c1_1_b), _rows(dn_c1_1_g), _rows(dn_c1_1_be)]
    x1t = _run_edge(xt0, x0, w_c1)

    x1 = jnp.transpose(x1t, (0, 2, 1))
    w_c2 = [dn_c2_0_W.T, _rows(dn_c2_0_b), _rows(dn_c2_0_g), _rows(dn_c2_0_be),
            dn_c2_1_W.T, _rows(dn_c2_1_b), _rows(dn_c2_1_g), _rows(dn_c2_1_be)]
    x2t = _run_edge(x1t, x1, w_c2)

    ct = jnp.concatenate([x1t, x2t], axis=2)
    utt = _run_batch(
        _mlp1_body,
        [ct, dn_c3_0_W.T, _rows(dn_c3_0_b), _rows(dn_c3_0_g), _rows(dn_c3_0_be),
         dn_c3_1_W.T, _rows(dn_c3_1_b), _rows(dn_c3_1_g), _rows(dn_c3_1_be)],
        1,
        [(N_DENSE, dn_c3_1_W.shape[0])],
    )

    zg = _run_batch(
        _ds_body,
        [utt, ds_W[:, :, 0, 0].T, ds_W[:, :, 0, 1].T, ds_W[:, :, 0, 2].T,
         _rows(ds_b), _rows(ds_g), _rows(ds_be)],
        1,
        [(N_DENSE, ds_W.shape[0])],
    )
    z4 = zg.reshape(bs, N_STROKE, N_PNT, -1)[:, :, ::2, :]
    ud = jnp.transpose(z4, (0, 3, 1, 2)).reshape(bs, -1, N_STROKE * (N_PNT // 2))
    return us, ud


# speedup vs baseline: 8.3102x; 2.4356x over previous
import jax
import jax.numpy as jnp
from jax.experimental import pallas as pl

N_STROKE = 32
N_PNT = 64
N_DENSE = N_STROKE * N_PNT
KNN = 10
ROW_TILE = 256
NEG = -3.0e38
_GELU_C = 0.7978845608028654


def _gelu(x):
    cdf = 0.5 * (1.0 + jnp.tanh(_GELU_C * (x + 0.044715 * (x * x * x))))
    return x * cdf


def _bn(x, g, be):
    return x / jnp.sqrt(jnp.float32(1.0 + 1e-5)) * g + be


def _mmb(a, b):
    return jax.lax.dot_general(a.astype(jnp.bfloat16), b.astype(jnp.bfloat16),
                               (((1,), (0,)), ((), ())),
                               preferred_element_type=jnp.float32)


def _mmb_nt(a, b):
    return jax.lax.dot_general(a.astype(jnp.bfloat16), b.astype(jnp.bfloat16),
                               (((1,), (1,)), ((), ())),
                               preferred_element_type=jnp.float32)


def _mm_hi(a, b):
    return jax.lax.dot_general(a, b, (((1,), (0,)), ((), ())),
                               precision=jax.lax.Precision.HIGHEST,
                               preferred_element_type=jnp.float32)


def _mm_raw(a, b):
    return jax.lax.dot_general(a, b, (((1,), (0,)), ((), ())),
                               preferred_element_type=jnp.float32)


def _split3(src):
    hi = src.astype(jnp.bfloat16)
    r1 = src - hi.astype(jnp.float32)
    mid = r1.astype(jnp.bfloat16)
    r2 = r1 - mid.astype(jnp.float32)
    lo = r2.astype(jnp.bfloat16)
    return hi, mid, lo


def _gather3(ek, parts):
    ekb = ek.astype(jnp.bfloat16)
    hi, mid, lo = parts
    return (_mm_raw(ekb, hi) + _mm_raw(ekb, mid)) + _mm_raw(ekb, lo)


def _shift_up(a, s):
    return jnp.concatenate([a[s:], a[:s]], axis=0)


def _shift_down(a, s):
    return jnp.concatenate([a[-s:], a[:-s]], axis=0)


def _topk_onehot_step(cur, iota_f):
    mx = jnp.max(cur, axis=1, keepdims=True)
    cand = jnp.where(cur == mx, iota_f, 3.0e38)
    amin = jnp.min(cand, axis=1, keepdims=True)
    hit = iota_f == amin
    return hit, jnp.where(hit, NEG, cur)


def _edge_mlp_step(ek, parts, xt, w1t, b1, g1, be1, w2t, b2, g2, be2):
    feat = _gather3(ek, parts)
    e = jnp.concatenate([feat - xt, xt], axis=1)
    h = _gelu(_bn(_mmb(e, w1t) + b1, g1, be1))
    return _gelu(_bn(_mmb(h, w2t) + b2, g2, be2))


def _knn_pd(xt, x):
    inner = -2.0 * _mmb(xt, x)
    xx_row = jnp.sum(x * x, axis=0, keepdims=True)
    xx_col = jnp.sum(xt * xt, axis=1, keepdims=True)
    return (-xx_col - inner) - xx_row


def _sparse_body(sft_ref, d4t_ref,
                 w0_ref, w1_ref, w2_ref, bcv_ref, gcv_ref, becv_ref,
                 w10_ref, b10_ref, g10_ref, be10_ref,
                 w11_ref, b11_ref, g11_ref, be11_ref,
                 w20_ref, b20_ref, g20_ref, be20_ref,
                 w21_ref, b21_ref, g21_ref, be21_ref,
                 w30_ref, b30_ref, g30_ref, be30_ref,
                 w31_ref, b31_ref, g31_ref, be31_ref,
                 out_ref):
    d4t = d4t_ref[0]
    a = (_mmb(d4t, w0_ref[...]) + _shift_up(_mmb(d4t, w1_ref[...]), 1)
         + _shift_up(_mmb(d4t, w2_ref[...]), 2)) + bcv_ref[...]
    z = _gelu(_bn(a, gcv_ref[...], becv_ref[...]))
    rowi = jax.lax.broadcasted_iota(jnp.int32, z.shape, 0)
    z = jnp.where((rowi & (N_PNT - 1)) < N_PNT - 2, z, NEG)
    m = z
    for s in (32, 16, 8, 4, 2, 1):
        m = jnp.maximum(m, _shift_up(m, s))
    si = jax.lax.broadcasted_iota(jnp.int32, (N_STROKE, N_DENSE), 0)
    ni = jax.lax.broadcasted_iota(jnp.int32, (N_STROKE, N_DENSE), 1)
    e1t = jnp.where(ni == si * N_PNT, 1.0, 0.0)
    ht = _gather3(e1t, _split3(m))
    x0 = jnp.concatenate([sft_ref[0], ht], axis=1)

    def gcn_block(xt, w1t, b1, g1, be1, w2t, b2, g2, be2):
        inner = -2.0 * _mmb_nt(xt, xt)
        sq = xt * xt
        xx_col = jnp.sum(sq, axis=1, keepdims=True)
        ones_row = jnp.ones((1, xt.shape[1]), jnp.float32)
        xx_row = jax.lax.dot_general(ones_row, sq, (((1,), (1,)), ((), ())),
                                     precision=jax.lax.Precision.HIGHEST,
                                     preferred_element_type=jnp.float32)
        pd = (-xx_col - inner) - xx_row
        iota_f = jax.lax.broadcasted_iota(jnp.int32, pd.shape, 1).astype(jnp.float32)
        cur = pd
        parts = _split3(xt)
        acc = jnp.full((xt.shape[0], w2t.shape[1]), NEG, jnp.float32)
        for _ in range(KNN):
            hit, cur = _topk_onehot_step(cur, iota_f)
            ek = jnp.where(hit, 1.0, 0.0)
            f = _edge_mlp_step(ek, parts, xt, w1t, b1, g1, be1, w2t, b2, g2, be2)
            acc = jnp.maximum(acc, f)
        return acc

    x1 = gcn_block(x0, w10_ref[...], b10_ref[...], g10_ref[...], be10_ref[...],
                   w11_ref[...], b11_ref[...], g11_ref[...], be11_ref[...])
    x2 = gcn_block(x1, w20_ref[...], b20_ref[...], g20_ref[...], be20_ref[...],
                   w21_ref[...], b21_ref[...], g21_ref[...], be21_ref[...])
    ct = jnp.concatenate([x1, x2], axis=1)
    h1 = _gelu(_bn(_mmb(ct, w30_ref[...]) + b30_ref[...],
                   g30_ref[...], be30_ref[...]))
    out_ref[0] = _gelu(_bn(_mmb(h1, w31_ref[...]) + b31_ref[...],
                           g31_ref[...], be31_ref[...]))


def _edge_body(xtf_ref, x_ref,
               w10_ref, b10_ref, g10_ref, be10_ref,
               w11_ref, b11_ref, g11_ref, be11_ref,
               out_ref):
    r = pl.program_id(1)
    xtf = xtf_ref[0]
    x = x_ref[0]
    xt = xtf_ref[0, pl.ds(r * ROW_TILE, ROW_TILE), :]
    pd = _knn_pd(xt, x)
    iota_f = jax.lax.broadcasted_iota(jnp.int32, pd.shape, 1).astype(jnp.float32)
    cur = pd
    parts = _split3(xtf)
    acc = jnp.full((ROW_TILE, w11_ref.shape[1]), NEG, jnp.float32)
    for _ in range(KNN):
        hit, cur = _topk_onehot_step(cur, iota_f)
        ek = jnp.where(hit, 1.0, 0.0)
        f = _edge_mlp_step(ek, parts, xt,
                           w10_ref[...], b10_ref[...], g10_ref[...], be10_ref[...],
                           w11_ref[...], b11_ref[...], g11_ref[...], be11_ref[...])
        acc = jnp.maximum(acc, f)
    out_ref[0] = acc


def _mlp1_body(ct_ref, w30_ref, b30_ref, g30_ref, be30_ref,
               w31_ref, b31_ref, g31_ref, be31_ref, out_ref):
    h1 = _gelu(_bn(_mmb(ct_ref[0], w30_ref[...]) + b30_ref[...],
                   g30_ref[...], be30_ref[...]))
    out_ref[0] = _gelu(_bn(_mmb(h1, w31_ref[...]) + b31_ref[...],
                           g31_ref[...], be31_ref[...]))


def _ds_body(ut_ref, w0_ref, w1_ref, w2_ref, b_ref, g_ref, be_ref, out_ref):
    ut = ut_ref[0]
    a0 = _mmb(ut, w0_ref[...])
    a1 = _mmb(ut, w1_ref[...])
    a2 = _mmb(ut, w2_ref[...])
    rowi = jax.lax.broadcasted_iota(jnp.int32, ut.shape, 0)
    pm = rowi & (N_PNT - 1)
    t0 = jnp.where(pm == 0, 0.0, _shift_down(a0, 1))
    t2 = jnp.where(pm == N_PNT - 1, 0.0, _shift_up(a2, 1))
    out_ref[0] = _gelu(_bn(t0 + a1 + t2 + b_ref[...], g_ref[...], be_ref[...]))


def _full_spec(shape):
    return pl.BlockSpec((1,) + shape[1:], lambda b, *_: (b,) + (0,) * (len(shape) - 1))


def _w_spec(shape):
    return pl.BlockSpec(shape, lambda *_: (0,) * len(shape))


def _run_batch(body, inputs, n_batched, out_shapes):
    bs = inputs[0].shape[0]
    in_specs = [_full_spec(a.shape) if i < n_batched else _w_spec(a.shape)
                for i, a in enumerate(inputs)]
    out_specs = [_full_spec((bs,) + s) for s in out_shapes]
    out_shape = [jax.ShapeDtypeStruct((bs,) + s, jnp.float32) for s in out_shapes]
    return pl.pallas_call(
        body,
        grid=(bs,),
        in_specs=in_specs,
        out_specs=out_specs if len(out_specs) > 1 else out_specs[0],
        out_shape=out_shape if len(out_shape) > 1 else out_shape[0],
    )(*inputs)


def _run_edge(xtf, x, weights):
    bs, n, c = xtf.shape
    c2 = weights[4].shape[1]
    ntile = n // ROW_TILE
    in_specs = [
        pl.BlockSpec((1, n, c), lambda b, r: (b, 0, 0)),
        pl.BlockSpec((1, c, n), lambda b, r: (b, 0, 0)),
    ] + [_w_spec(w.shape) for w in weights]
    return pl.pallas_call(
        _edge_body,
        grid=(bs, ntile),
        in_specs=in_specs,
        out_specs=pl.BlockSpec((1, ROW_TILE, c2), lambda b, r: (b, r, 0)),
        out_shape=jax.ShapeDtypeStruct((bs, n, c2), jnp.float32),
    )(xtf, x, *weights)


def _rows(v):
    return v[None, :]


def kernel(sparse_fea, dense_fea, sp_c1_0_W, sp_c1_0_b, sp_c1_0_g, sp_c1_0_be, sp_c1_1_W, sp_c1_1_b, sp_c1_1_g, sp_c1_1_be, sp_c2_0_W, sp_c2_0_b, sp_c2_0_g, sp_c2_0_be, sp_c2_1_W, sp_c2_1_b, sp_c2_1_g, sp_c2_1_be, sp_c3_0_W, sp_c3_0_b, sp_c3_0_g, sp_c3_0_be, sp_c3_1_W, sp_c3_1_b, sp_c3_1_g, sp_c3_1_be, dn_c1_0_W, dn_c1_0_b, dn_c1_0_g, dn_c1_0_be, dn_c1_1_W, dn_c1_1_b, dn_c1_1_g, dn_c1_1_be, dn_c2_0_W, dn_c2_0_b, dn_c2_0_g, dn_c2_0_be, dn_c2_1_W, dn_c2_1_b, dn_c2_1_g, dn_c2_1_be, dn_c3_0_W, dn_c3_0_b, dn_c3_0_g, dn_c3_0_be, dn_c3_1_W, dn_c3_1_b, dn_c3_1_g, dn_c3_1_be, d2s_W, d2s_b, d2s_g, d2s_be, ds_W, ds_b, ds_g, ds_be):
    bs = sparse_fea.shape[0]

    sft = jnp.transpose(sparse_fea, (0, 2, 1))
    d4t = jnp.transpose(dense_fea, (0, 2, 1))

    ust = _run_batch(
        _sparse_body,
        [sft, d4t,
         d2s_W[:, :, 0, 0].T, d2s_W[:, :, 0, 1].T, d2s_W[:, :, 0, 2].T,
         _rows(d2s_b), _rows(d2s_g), _rows(d2s_be),
         sp_c1_0_W.T, _rows(sp_c1_0_b), _rows(sp_c1_0_g), _rows(sp_c1_0_be),
         sp_c1_1_W.T, _rows(sp_c1_1_b), _rows(sp_c1_1_g), _rows(sp_c1_1_be),
         sp_c2_0_W.T, _rows(sp_c2_0_b), _rows(sp_c2_0_g), _rows(sp_c2_0_be),
         sp_c2_1_W.T, _rows(sp_c2_1_b), _rows(sp_c2_1_g), _rows(sp_c2_1_be),
         sp_c3_0_W.T, _rows(sp_c3_0_b), _rows(sp_c3_0_g), _rows(sp_c3_0_be),
         sp_c3_1_W.T, _rows(sp_c3_1_b), _rows(sp_c3_1_g), _rows(sp_c3_1_be)],
        2,
        [(N_STROKE, sp_c3_1_W.shape[0])],
    )
    us = jnp.transpose(ust, (0, 2, 1))

    sp_rep_t = jnp.repeat(sft, N_PNT, axis=1)
    xt0 = jnp.concatenate([d4t, sp_rep_t], axis=2)
    x0 = jnp.transpose(xt0, (0, 2, 1))

    w_c1 = [dn_c1_0_W.T, _rows(dn_c1_0_b), _rows(dn_c1_0_g), _rows(dn_c1_0_be),
            dn_c1_1_W.T, _rows(dn_c1_1_b), _rows(dn_c1_1_g), _rows(dn_c1_1_be)]
    x1t = _run_edge(xt0, x0, w_c1)

    x1 = jnp.transpose(x1t, (0, 2, 1))
    w_c2 = [dn_c2_0_W.T, _rows(dn_c2_0_b), _rows(dn_c2_0_g), _rows(dn_c2_0_be),
            dn_c2_1_W.T, _rows(dn_c2_1_b), _rows(dn_c2_1_g), _rows(dn_c2_1_be)]
    x2t = _run_edge(x1t, x1, w_c2)

    ct = jnp.concatenate([x1t, x2t], axis=2)
    utt = _run_batch(
        _mlp1_body,
        [ct, dn_c3_0_W.T, _rows(dn_c3_0_b), _rows(dn_c3_0_g), _rows(dn_c3_0_be),
         dn_c3_1_W.T, _rows(dn_c3_1_b), _rows(dn_c3_1_g), _rows(dn_c3_1_be)],
        1,
        [(N_DENSE, dn_c3_1_W.shape[0])],
    )

    zg = _run_batch(
        _ds_body,
        [utt, ds_W[:, :, 0, 0].T, ds_W[:, :, 0, 1].T, ds_W[:, :, 0, 2].T,
         _rows(ds_b), _rows(ds_g), _rows(ds_be)],
        1,
        [(N_DENSE, ds_W.shape[0])],
    )
    z4 = zg.reshape(bs, N_STROKE, N_PNT, -1)[:, :, ::2, :]
    ud = jnp.transpose(z4, (0, 3, 1, 2)).reshape(bs, -1, N_STROKE * (N_PNT // 2))
    return us, ud


# speedup vs baseline: 10.5536x; 1.2700x over previous
import jax
import jax.numpy as jnp
from jax.experimental import pallas as pl

N_STROKE = 32
N_PNT = 64
N_DENSE = N_STROKE * N_PNT
KNN = 10
ROW_TILE = 512
NEG = -3.0e38
_GELU_C = 0.7978845608028654


def _gelu(x):
    cdf = 0.5 * (1.0 + jnp.tanh(_GELU_C * (x + 0.044715 * (x * x * x))))
    return x * cdf


def _bn(x, g, be):
    return x / jnp.sqrt(jnp.float32(1.0 + 1e-5)) * g + be


def _mmb(a, b):
    return jax.lax.dot_general(a.astype(jnp.bfloat16), b.astype(jnp.bfloat16),
                               (((1,), (0,)), ((), ())),
                               preferred_element_type=jnp.float32)


def _mmb_nt(a, b):
    return jax.lax.dot_general(a.astype(jnp.bfloat16), b.astype(jnp.bfloat16),
                               (((1,), (1,)), ((), ())),
                               preferred_element_type=jnp.float32)


def _mm_hi(a, b):
    return jax.lax.dot_general(a, b, (((1,), (0,)), ((), ())),
                               precision=jax.lax.Precision.HIGHEST,
                               preferred_element_type=jnp.float32)


def _mm_raw(a, b):
    return jax.lax.dot_general(a, b, (((1,), (0,)), ((), ())),
                               preferred_element_type=jnp.float32)


def _split3(src):
    hi = src.astype(jnp.bfloat16)
    r1 = src - hi.astype(jnp.float32)
    mid = r1.astype(jnp.bfloat16)
    r2 = r1 - mid.astype(jnp.float32)
    lo = r2.astype(jnp.bfloat16)
    return jnp.concatenate([hi, mid, lo], axis=1)


def _gather3(ek, parts):
    c = parts.shape[1] // 3
    g = _mm_raw(ek.astype(jnp.bfloat16), parts)
    return (g[:, :c] + g[:, c:2 * c]) + g[:, 2 * c:]


def _shift_up(a, s):
    return jnp.concatenate([a[s:], a[:s]], axis=0)


def _shift_down(a, s):
    return jnp.concatenate([a[-s:], a[:-s]], axis=0)


def _topk_onehot_step(cur, iota_f):
    mx = jnp.max(cur, axis=1, keepdims=True)
    cand = jnp.where(cur == mx, iota_f, 3.0e38)
    amin = jnp.min(cand, axis=1, keepdims=True)
    hit = iota_f == amin
    return hit, jnp.where(hit, NEG, cur)


def _edge_mlp_step(ek, parts, xt, w1t, b1, g1, be1, w2t, b2, g2, be2):
    feat = _gather3(ek, parts)
    e = jnp.concatenate([feat - xt, xt], axis=1)
    h = _gelu(_bn(_mmb(e, w1t) + b1, g1, be1))
    return _gelu(_bn(_mmb(h, w2t) + b2, g2, be2))


def _knn_pd(xt, x):
    inner = -2.0 * _mmb(xt, x)
    xx_row = jnp.sum(x * x, axis=0, keepdims=True)
    xx_col = jnp.sum(xt * xt, axis=1, keepdims=True)
    return (-xx_col - inner) - xx_row


def _sparse_body(sft_ref, d4t_ref,
                 w0_ref, w1_ref, w2_ref, bcv_ref, gcv_ref, becv_ref,
                 w10_ref, b10_ref, g10_ref, be10_ref,
                 w11_ref, b11_ref, g11_ref, be11_ref,
                 w20_ref, b20_ref, g20_ref, be20_ref,
                 w21_ref, b21_ref, g21_ref, be21_ref,
                 w30_ref, b30_ref, g30_ref, be30_ref,
                 w31_ref, b31_ref, g31_ref, be31_ref,
                 out_ref):
    d4t = d4t_ref[0]
    a = (_mmb(d4t, w0_ref[...]) + _shift_up(_mmb(d4t, w1_ref[...]), 1)
         + _shift_up(_mmb(d4t, w2_ref[...]), 2)) + bcv_ref[...]
    z = _gelu(_bn(a, gcv_ref[...], becv_ref[...]))
    rowi = jax.lax.broadcasted_iota(jnp.int32, z.shape, 0)
    z = jnp.where((rowi & (N_PNT - 1)) < N_PNT - 2, z, NEG)
    m = z
    for s in (32, 16, 8, 4, 2, 1):
        m = jnp.maximum(m, _shift_up(m, s))
    si = jax.lax.broadcasted_iota(jnp.int32, (N_STROKE, N_DENSE), 0)
    ni = jax.lax.broadcasted_iota(jnp.int32, (N_STROKE, N_DENSE), 1)
    e1t = jnp.where(ni == si * N_PNT, 1.0, 0.0)
    ht = _gather3(e1t, _split3(m))
    x0 = jnp.concatenate([sft_ref[0], ht], axis=1)

    def gcn_block(xt, w1t, b1, g1, be1, w2t, b2, g2, be2):
        inner = -2.0 * _mmb_nt(xt, xt)
        sq = xt * xt
        xx_col = jnp.sum(sq, axis=1, keepdims=True)
        ones_row = jnp.ones((1, xt.shape[1]), jnp.float32)
        xx_row = jax.lax.dot_general(ones_row, sq, (((1,), (1,)), ((), ())),
                                     precision=jax.lax.Precision.HIGHEST,
                                     preferred_element_type=jnp.float32)
        pd = (-xx_col - inner) - xx_row
        iota_f = jax.lax.broadcasted_iota(jnp.int32, pd.shape, 1).astype(jnp.float32)
        cur = pd
        parts = _split3(xt)
        acc = jnp.full((xt.shape[0], w2t.shape[1]), NEG, jnp.float32)
        for _ in range(KNN):
            hit, cur = _topk_onehot_step(cur, iota_f)
            ek = jnp.where(hit, 1.0, 0.0)
            f = _edge_mlp_step(ek, parts, xt, w1t, b1, g1, be1, w2t, b2, g2, be2)
            acc = jnp.maximum(acc, f)
        return acc

    x1 = gcn_block(x0, w10_ref[...], b10_ref[...], g10_ref[...], be10_ref[...],
                   w11_ref[...], b11_ref[...], g11_ref[...], be11_ref[...])
    x2 = gcn_block(x1, w20_ref[...], b20_ref[...], g20_ref[...], be20_ref[...],
                   w21_ref[...], b21_ref[...], g21_ref[...], be21_ref[...])
    ct = jnp.concatenate([x1, x2], axis=1)
    h1 = _gelu(_bn(_mmb(ct, w30_ref[...]) + b30_ref[...],
                   g30_ref[...], be30_ref[...]))
    out_ref[0] = _gelu(_bn(_mmb(h1, w31_ref[...]) + b31_ref[...],
                           g31_ref[...], be31_ref[...]))


def _edge_body(xtf_ref, x_ref,
               w10_ref, b10_ref, g10_ref, be10_ref,
               w11_ref, b11_ref, g11_ref, be11_ref,
               out_ref):
    r = pl.program_id(1)
    xtf = xtf_ref[0]
    x = x_ref[0]
    xt = xtf_ref[0, pl.ds(r * ROW_TILE, ROW_TILE), :]
    pd = _knn_pd(xt, x)
    iota_f = jax.lax.broadcasted_iota(jnp.int32, pd.shape, 1).astype(jnp.float32)
    cur = pd
    parts = _split3(xtf)
    acc = jnp.full((ROW_TILE, w11_ref.shape[1]), NEG, jnp.float32)
    for _ in range(KNN):
        hit, cur = _topk_onehot_step(cur, iota_f)
        ek = jnp.where(hit, 1.0, 0.0)
        f = _edge_mlp_step(ek, parts, xt,
                           w10_ref[...], b10_ref[...], g10_ref[...], be10_ref[...],
                           w11_ref[...], b11_ref[...], g11_ref[...], be11_ref[...])
        acc = jnp.maximum(acc, f)
    out_ref[0] = acc


def _mlp1_body(ct_ref, w30_ref, b30_ref, g30_ref, be30_ref,
               w31_ref, b31_ref, g31_ref, be31_ref, out_ref):
    h1 = _gelu(_bn(_mmb(ct_ref[0], w30_ref[...]) + b30_ref[...],
                   g30_ref[...], be30_ref[...]))
    out_ref[0] = _gelu(_bn(_mmb(h1, w31_ref[...]) + b31_ref[...],
                           g31_ref[...], be31_ref[...]))


def _ds_body(ut_ref, w0_ref, w1_ref, w2_ref, b_ref, g_ref, be_ref, out_ref):
    ut = ut_ref[0]
    a0 = _mmb(ut, w0_ref[...])
    a1 = _mmb(ut, w1_ref[...])
    a2 = _mmb(ut, w2_ref[...])
    rowi = jax.lax.broadcasted_iota(jnp.int32, ut.shape, 0)
    pm = rowi & (N_PNT - 1)
    t0 = jnp.where(pm == 0, 0.0, _shift_down(a0, 1))
    t2 = jnp.where(pm == N_PNT - 1, 0.0, _shift_up(a2, 1))
    out_ref[0] = _gelu(_bn(t0 + a1 + t2 + b_ref[...], g_ref[...], be_ref[...]))


def _full_spec(shape):
    return pl.BlockSpec((1,) + shape[1:], lambda b, *_: (b,) + (0,) * (len(shape) - 1))


def _w_spec(shape):
    return pl.BlockSpec(shape, lambda *_: (0,) * len(shape))


def _run_batch(body, inputs, n_batched, out_shapes):
    bs = inputs[0].shape[0]
    in_specs = [_full_spec(a.shape) if i < n_batched else _w_spec(a.shape)
                for i, a in enumerate(inputs)]
    out_specs = [_full_spec((bs,) + s) for s in out_shapes]
    out_shape = [jax.ShapeDtypeStruct((bs,) + s, jnp.float32) for s in out_shapes]
    return pl.pallas_call(
        body,
        grid=(bs,),
        in_specs=in_specs,
        out_specs=out_specs if len(out_specs) > 1 else out_specs[0],
        out_shape=out_shape if len(out_shape) > 1 else out_shape[0],
    )(*inputs)


def _run_edge(xtf, x, weights):
    bs, n, c = xtf.shape
    c2 = weights[4].shape[1]
    ntile = n // ROW_TILE
    in_specs = [
        pl.BlockSpec((1, n, c), lambda b, r: (b, 0, 0)),
        pl.BlockSpec((1, c, n), lambda b, r: (b, 0, 0)),
    ] + [_w_spec(w.shape) for w in weights]
    return pl.pallas_call(
        _edge_body,
        grid=(bs, ntile),
        in_specs=in_specs,
        out_specs=pl.BlockSpec((1, ROW_TILE, c2), lambda b, r: (b, r, 0)),
        out_shape=jax.ShapeDtypeStruct((bs, n, c2), jnp.float32),
    )(xtf, x, *weights)


def _rows(v):
    return v[None, :]


def kernel(sparse_fea, dense_fea, sp_c1_0_W, sp_c1_0_b, sp_c1_0_g, sp_c1_0_be, sp_c1_1_W, sp_c1_1_b, sp_c1_1_g, sp_c1_1_be, sp_c2_0_W, sp_c2_0_b, sp_c2_0_g, sp_c2_0_be, sp_c2_1_W, sp_c2_1_b, sp_c2_1_g, sp_c2_1_be, sp_c3_0_W, sp_c3_0_b, sp_c3_0_g, sp_c3_0_be, sp_c3_1_W, sp_c3_1_b, sp_c3_1_g, sp_c3_1_be, dn_c1_0_W, dn_c1_0_b, dn_c1_0_g, dn_c1_0_be, dn_c1_1_W, dn_c1_1_b, dn_c1_1_g, dn_c1_1_be, dn_c2_0_W, dn_c2_0_b, dn_c2_0_g, dn_c2_0_be, dn_c2_1_W, dn_c2_1_b, dn_c2_1_g, dn_c2_1_be, dn_c3_0_W, dn_c3_0_b, dn_c3_0_g, dn_c3_0_be, dn_c3_1_W, dn_c3_1_b, dn_c3_1_g, dn_c3_1_be, d2s_W, d2s_b, d2s_g, d2s_be, ds_W, ds_b, ds_g, ds_be):
    bs = sparse_fea.shape[0]

    sft = jnp.transpose(sparse_fea, (0, 2, 1))
    d4t = jnp.transpose(dense_fea, (0, 2, 1))

    ust = _run_batch(
        _sparse_body,
        [sft, d4t,
         d2s_W[:, :, 0, 0].T, d2s_W[:, :, 0, 1].T, d2s_W[:, :, 0, 2].T,
         _rows(d2s_b), _rows(d2s_g), _rows(d2s_be),
         sp_c1_0_W.T, _rows(sp_c1_0_b), _rows(sp_c1_0_g), _rows(sp_c1_0_be),
         sp_c1_1_W.T, _rows(sp_c1_1_b), _rows(sp_c1_1_g), _rows(sp_c1_1_be),
         sp_c2_0_W.T, _rows(sp_c2_0_b), _rows(sp_c2_0_g), _rows(sp_c2_0_be),
         sp_c2_1_W.T, _rows(sp_c2_1_b), _rows(sp_c2_1_g), _rows(sp_c2_1_be),
         sp_c3_0_W.T, _rows(sp_c3_0_b), _rows(sp_c3_0_g), _rows(sp_c3_0_be),
         sp_c3_1_W.T, _rows(sp_c3_1_b), _rows(sp_c3_1_g), _rows(sp_c3_1_be)],
        2,
        [(N_STROKE, sp_c3_1_W.shape[0])],
    )
    us = jnp.transpose(ust, (0, 2, 1))

    sp_rep_t = jnp.repeat(sft, N_PNT, axis=1)
    xt0 = jnp.concatenate([d4t, sp_rep_t], axis=2)
    x0 = jnp.transpose(xt0, (0, 2, 1))

    w_c1 = [dn_c1_0_W.T, _rows(dn_c1_0_b), _rows(dn_c1_0_g), _rows(dn_c1_0_be),
            dn_c1_1_W.T, _rows(dn_c1_1_b), _rows(dn_c1_1_g), _rows(dn_c1_1_be)]
    x1t = _run_edge(xt0, x0, w_c1)

    x1 = jnp.transpose(x1t, (0, 2, 1))
    w_c2 = [dn_c2_0_W.T, _rows(dn_c2_0_b), _rows(dn_c2_0_g), _rows(dn_c2_0_be),
            dn_c2_1_W.T, _rows(dn_c2_1_b), _rows(dn_c2_1_g), _rows(dn_c2_1_be)]
    x2t = _run_edge(x1t, x1, w_c2)

    ct = jnp.concatenate([x1t, x2t], axis=2)
    utt = _run_batch(
        _mlp1_body,
        [ct, dn_c3_0_W.T, _rows(dn_c3_0_b), _rows(dn_c3_0_g), _rows(dn_c3_0_be),
         dn_c3_1_W.T, _rows(dn_c3_1_b), _rows(dn_c3_1_g), _rows(dn_c3_1_be)],
        1,
        [(N_DENSE, dn_c3_1_W.shape[0])],
    )

    zg = _run_batch(
        _ds_body,
        [utt, ds_W[:, :, 0, 0].T, ds_W[:, :, 0, 1].T, ds_W[:, :, 0, 2].T,
         _rows(ds_b), _rows(ds_g), _rows(ds_be)],
        1,
        [(N_DENSE, ds_W.shape[0])],
    )
    z4 = zg.reshape(bs, N_STROKE, N_PNT, -1)[:, :, ::2, :]
    ud = jnp.transpose(z4, (0, 3, 1, 2)).reshape(bs, -1, N_STROKE * (N_PNT // 2))
    return us, ud
